# Initial kernel scaffold; baseline (speedup 1.0000x reference)
import functools

import jax
import jax.numpy as jnp
from jax import lax
from jax.experimental import pallas as pl
from jax.experimental.pallas import tpu as pltpu
from jax.experimental.pallas import tpu_sc as plsc

N = 10000
NPAD = 10240
NC, NS = 2, 16
NW = NC * NS
EK = 128
ROWS_PER_W = NPAD // NW
ROWS_PER_S = NPAD // NS
XCH, XK = 4, 80


def _mesh():
    return plsc.VectorSubcoreMesh(core_axis_name="c", subcore_axis_name="s")


def _make_prep(ech):
    @functools.partial(
        pl.kernel,
        out_type=[
            jax.ShapeDtypeStruct((NW, NPAD), jnp.float32),
            jax.ShapeDtypeStruct((NPAD, 128), jnp.float32),
        ],
        mesh=_mesh(),
        scratch_types=[
            pltpu.VMEM((ech, EK), jnp.int32),
            pltpu.VMEM((ech, EK), jnp.float32),
            pltpu.VMEM((XCH, XK), jnp.int32),
            pltpu.VMEM((XK, 128), jnp.float32),
            pltpu.VMEM((NPAD,), jnp.float32),
            pltpu.SemaphoreType.DMA,
        ],
    )
    def prep(col_hbm, ew_hbm, x_hbm, emb_hbm, deg_out, h0_out,
             colv, ewv, xv, ebuf, degv, sem):
        cc = lax.axis_index("c")
        ss = lax.axis_index("s")
        wid = cc * NS + ss
        pltpu.sync_copy(col_hbm.at[wid], colv)
        pltpu.sync_copy(ew_hbm.at[wid], ewv)
        pltpu.sync_copy(x_hbm.at[wid], xv)

        def zero_body(i, _):
            degv[pl.ds(i * 16, 16)] = jnp.zeros((16,), jnp.float32)
            return 0

        lax.fori_loop(0, NPAD // 16, zero_body, 0)

        def hist_body(j, _):
            for k in range(EK // 16):
                c16 = colv[j, pl.ds(k * 16, 16)]
                w16 = ewv[j, pl.ds(k * 16, 16)]
                plsc.addupdate_scatter(degv, [c16], w16)
            return 0

        lax.fori_loop(0, ech, hist_body, 0)
        pltpu.sync_copy(degv, deg_out.at[wid])

        for j in range(XCH):
            pltpu.async_copy(emb_hbm.at[xv.at[j]], ebuf, sem).wait()
            pltpu.sync_copy(
                ebuf, h0_out.at[pl.ds(wid * ROWS_PER_W + j * XK, XK)])

    return prep


def _make_agg(ech, d):
    @functools.partial(
        pl.kernel,
        out_type=jax.ShapeDtypeStruct((NC, NPAD, d), jnp.float32),
        mesh=_mesh(),
        scratch_types=[
            pltpu.VMEM((ech, EK), jnp.int32),
            pltpu.VMEM((ech, EK), jnp.int32),
            pltpu.VMEM((ech, EK), jnp.float32),
            pltpu.VMEM((EK, d), jnp.float32),
            pltpu.VMEM_SHARED((NPAD, d), jnp.float32),
            pltpu.SemaphoreType.DMA,
        ],
    )
    def agg(g_hbm, row_hbm, col_hbm, ew_hbm, out_hbm,
            rowv, colv, ewv, buf, acc, sem):
        cc = lax.axis_index("c")
        ss = lax.axis_index("s")
        wid = cc * NS + ss
        pltpu.sync_copy(row_hbm.at[wid], rowv)
        pltpu.sync_copy(col_hbm.at[wid], colv)
        pltpu.sync_copy(ew_hbm.at[wid], ewv)

        def zero_body(i, _):
            for t in range(d // 16):
                buf[i, pl.ds(t * 16, 16)] = jnp.zeros((16,), jnp.float32)
            return 0

        lax.fori_loop(0, EK, zero_body, 0)
        for t in range(ROWS_PER_S // EK):
            pltpu.sync_copy(buf, acc.at[pl.ds(ss * ROWS_PER_S + t * EK, EK)])
        plsc.subcore_barrier()

        def chunk_body(j, _):
            pltpu.async_copy(g_hbm.at[rowv.at[j]], buf, sem).wait()

            def scale_body(k, _):
                w = plsc.load_gather(
                    ewv,
                    [jnp.broadcast_to(j, (16,)).astype(jnp.int32),
                     jnp.broadcast_to(k, (16,)).astype(jnp.int32)])
                for t in range(d // 16):
                    sl = pl.ds(t * 16, 16)
                    buf[k, sl] = buf[k, sl] * w
                return 0

            lax.fori_loop(0, EK, scale_body, 0)
            pltpu.sync_copy(buf, acc.at[colv.at[j]], add=True)
            return 0

        lax.fori_loop(0, ech, chunk_body, 0)
        plsc.subcore_barrier()

        for t in range(ROWS_PER_S // EK):
            base = ss * ROWS_PER_S + t * EK
            pltpu.sync_copy(acc.at[pl.ds(base, EK)], buf)
            pltpu.sync_copy(buf, out_hbm.at[cc, pl.ds(base, EK)])

    return agg


def _dinv_and_mask(degp):
    deg = jnp.sum(degp, axis=0) + 1.0
    mask = lax.broadcasted_iota(jnp.int32, (NPAD, 1), 0) < N
    dinv = jnp.where(mask[:, 0], lax.rsqrt(deg), 0.0)
    return dinv, mask


def _tc1_body(degp_ref, h0_ref, w1_ref, g1_ref):
    dinv, _ = _dinv_and_mask(degp_ref[...])
    hw = jnp.dot(h0_ref[...], w1_ref[...], preferred_element_type=jnp.float32)
    g1_ref[...] = hw * dinv[:, None]


def _tc2_body(aggp_ref, g1_ref, degp_ref, b_ref, gm_ref, bt_ref, w2_ref,
              g2_ref):
    dinv, mask = _dinv_and_mask(degp_ref[...])
    agg = aggp_ref[0] + aggp_ref[1]
    pre = (agg + g1_ref[...]) * dinv[:, None] + b_ref[...][None, :]
    cnt = jnp.float32(N)
    mean = jnp.sum(jnp.where(mask, pre, 0.0), axis=0, keepdims=True) / cnt
    dev = jnp.where(mask, pre - mean, 0.0)
    var = jnp.sum(dev * dev, axis=0, keepdims=True) / cnt
    h1 = (pre - mean) * lax.rsqrt(var + 1e-5) * gm_ref[...][None, :] \
        + bt_ref[...][None, :]
    h1 = jnp.where(mask, jnp.maximum(h1, 0.0), 0.0)
    hw2 = jnp.dot(h1, w2_ref[...], preferred_element_type=jnp.float32)
    g2_ref[...] = hw2 * dinv[:, None]


def _tc3_body(aggp_ref, g2_ref, degp_ref, b_ref, gm_ref, bt_ref, out_ref):
    dinv, mask = _dinv_and_mask(degp_ref[...])
    agg = aggp_ref[0] + aggp_ref[1]
    pre = (agg + g2_ref[...]) * dinv[:, None] + b_ref[...][None, :]
    cnt = jnp.float32(N)
    mean = jnp.sum(jnp.where(mask, pre, 0.0), axis=0, keepdims=True) / cnt
    dev = jnp.where(mask, pre - mean, 0.0)
    var = jnp.sum(dev * dev, axis=0, keepdims=True) / cnt
    out_ref[...] = (pre - mean) * lax.rsqrt(var + 1e-5) * gm_ref[...][None, :] \
        + bt_ref[...][None, :]


def kernel(x, edge_index, weight, emb_table, W1, b1, gamma1, beta1,
           W2, b2, gamma2, beta2):
    x = x.astype(jnp.int32)
    row = edge_index[0].astype(jnp.int32)
    col = edge_index[1].astype(jnp.int32)
    e = row.shape[0]
    per_w = -(-e // NW)
    ech = -(-per_w // EK)
    epad = NW * ech * EK
    rowp = jnp.pad(row, (0, epad - e)).reshape(NW, ech, EK)
    colp = jnp.pad(col, (0, epad - e)).reshape(NW, ech, EK)
    ewp = jnp.pad(weight, (0, epad - e)).reshape(NW, ech, EK)
    xp = jnp.pad(x, (0, NPAD - N)).reshape(NW, XCH, XK)

    degp, h0 = _make_prep(ech)(colp, ewp, xp, emb_table)

    g1 = pl.pallas_call(
        _tc1_body,
        out_shape=jax.ShapeDtypeStruct((NPAD, 128), jnp.float32),
    )(degp, h0, W1)

    agg1p = _make_agg(ech, 128)(g1, rowp, colp, ewp)

    g2 = pl.pallas_call(
        _tc2_body,
        out_shape=jax.ShapeDtypeStruct((NPAD, 64), jnp.float32),
    )(agg1p, g1, degp, b1, gamma1, beta1, W2)

    agg2p = _make_agg(ech, 64)(g2, rowp, colp, ewp)

    out = pl.pallas_call(
        _tc3_body,
        out_shape=jax.ShapeDtypeStruct((NPAD, 64), jnp.float32),
    )(agg2p, g2, degp, b2, gamma2, beta2)

    return out[:N]


# speedup vs baseline: 12.3366x; 12.3366x over previous
import functools

import jax
import jax.numpy as jnp
from jax import lax
from jax.experimental import pallas as pl
from jax.experimental.pallas import tpu as pltpu
from jax.experimental.pallas import tpu_sc as plsc

N = 10000
NPAD = 10240
NC, NS = 2, 16
NW = NC * NS
EK = 128
ROWS_PER_W = NPAD // NW
ROWS_PER_S = NPAD // NS
XCH, XK = 4, 80


def _mesh():
    return plsc.VectorSubcoreMesh(core_axis_name="c", subcore_axis_name="s")


def _make_prep(ech):
    @functools.partial(
        pl.kernel,
        out_type=[
            jax.ShapeDtypeStruct((NW, NPAD), jnp.float32),
            jax.ShapeDtypeStruct((NPAD, 128), jnp.float32),
        ],
        mesh=_mesh(),
        scratch_types=[
            pltpu.VMEM((ech, EK), jnp.int32),
            pltpu.VMEM((ech, EK), jnp.float32),
            pltpu.VMEM((XCH, XK), jnp.int32),
            pltpu.VMEM((XK, 128), jnp.float32),
            pltpu.VMEM((NPAD,), jnp.float32),
            pltpu.SemaphoreType.DMA,
        ],
        compiler_params=pltpu.CompilerParams(needs_layout_passes=False),
    )
    def prep(col_hbm, ew_hbm, x_hbm, emb_hbm, deg_out, h0_out,
             colv, ewv, xv, ebuf, degv, sem):
        cc = lax.axis_index("c")
        ss = lax.axis_index("s")
        wid = cc * NS + ss
        pltpu.sync_copy(col_hbm.at[wid], colv)
        pltpu.sync_copy(ew_hbm.at[wid], ewv)
        pltpu.sync_copy(x_hbm.at[wid], xv)

        def zero_body(i, _):
            degv[pl.ds(i * 16, 16)] = jnp.zeros((16,), jnp.float32)
            return 0

        lax.fori_loop(0, NPAD // 16, zero_body, 0)

        def hist_body(j, _):
            for k in range(EK // 16):
                c16 = colv[j, pl.ds(k * 16, 16)]
                w16 = ewv[j, pl.ds(k * 16, 16)]
                plsc.addupdate_scatter(degv, [c16], w16)
            return 0

        lax.fori_loop(0, ech, hist_body, 0)
        pltpu.sync_copy(degv, deg_out.at[wid])

        for j in range(XCH):
            pltpu.async_copy(emb_hbm.at[xv.at[j]], ebuf, sem).wait()
            pltpu.sync_copy(
                ebuf, h0_out.at[pl.ds(wid * ROWS_PER_W + j * XK, XK)])

    return prep


def _make_agg(ech, d):
    @functools.partial(
        pl.kernel,
        out_type=jax.ShapeDtypeStruct((NC, NPAD, d), jnp.float32),
        mesh=_mesh(),
        scratch_types=[
            pltpu.VMEM((ech, EK), jnp.int32),
            pltpu.VMEM((ech, EK), jnp.int32),
            pltpu.VMEM((ech, EK), jnp.float32),
            pltpu.VMEM((EK, d), jnp.float32),
            pltpu.VMEM_SHARED((NPAD, d), jnp.float32),
            pltpu.SemaphoreType.DMA,
        ],
        compiler_params=pltpu.CompilerParams(
            needs_layout_passes=False, use_tc_tiling_on_sc=False),
    )
    def agg(g_hbm, row_hbm, col_hbm, ew_hbm, out_hbm,
            rowv, colv, ewv, buf, acc, sem):
        cc = lax.axis_index("c")
        ss = lax.axis_index("s")
        wid = cc * NS + ss
        pltpu.sync_copy(row_hbm.at[wid], rowv)
        pltpu.sync_copy(col_hbm.at[wid], colv)
        pltpu.sync_copy(ew_hbm.at[wid], ewv)

        def zero_body(i, _):
            for t in range(d // 16):
                buf[i, pl.ds(t * 16, 16)] = jnp.zeros((16,), jnp.float32)
            return 0

        lax.fori_loop(0, EK, zero_body, 0)
        for t in range(ROWS_PER_S // EK):
            pltpu.sync_copy(buf, acc.at[pl.ds(ss * ROWS_PER_S + t * EK, EK)])
        plsc.subcore_barrier()

        def chunk_body(j, _):
            pltpu.async_copy(g_hbm.at[rowv.at[j]], buf, sem).wait()

            def scale_body(k, _):
                w = plsc.load_gather(
                    ewv,
                    [jnp.broadcast_to(j, (16,)).astype(jnp.int32),
                     jnp.broadcast_to(k, (16,)).astype(jnp.int32)])
                for t in range(d // 16):
                    sl = pl.ds(t * 16, 16)
                    buf[k, sl] = buf[k, sl] * w
                return 0

            lax.fori_loop(0, EK, scale_body, 0)
            pltpu.sync_copy(buf, acc.at[colv.at[j]], add=True)
            return 0

        lax.fori_loop(0, ech, chunk_body, 0)
        plsc.subcore_barrier()

        for t in range(ROWS_PER_S // EK):
            base = ss * ROWS_PER_S + t * EK
            pltpu.sync_copy(acc.at[pl.ds(base, EK)], buf)
            pltpu.sync_copy(buf, out_hbm.at[cc, pl.ds(base, EK)])

    return agg


def _dinv_and_mask(degp):
    deg = jnp.sum(degp, axis=0) + 1.0
    mask = lax.broadcasted_iota(jnp.int32, (NPAD, 1), 0) < N
    dinv = jnp.where(mask[:, 0], lax.rsqrt(deg), 0.0)
    return dinv, mask


def _tc1_body(degp_ref, h0_ref, w1_ref, g1_ref):
    dinv, _ = _dinv_and_mask(degp_ref[...])
    hw = jnp.dot(h0_ref[...], w1_ref[...], preferred_element_type=jnp.float32)
    g1_ref[...] = hw * dinv[:, None]


def _tc2_body(aggp_ref, g1_ref, degp_ref, b_ref, gm_ref, bt_ref, w2_ref,
              g2_ref):
    dinv, mask = _dinv_and_mask(degp_ref[...])
    agg = aggp_ref[0] + aggp_ref[1]
    pre = (agg + g1_ref[...]) * dinv[:, None] + b_ref[...][None, :]
    cnt = jnp.float32(N)
    mean = jnp.sum(jnp.where(mask, pre, 0.0), axis=0, keepdims=True) / cnt
    dev = jnp.where(mask, pre - mean, 0.0)
    var = jnp.sum(dev * dev, axis=0, keepdims=True) / cnt
    h1 = (pre - mean) * lax.rsqrt(var + 1e-5) * gm_ref[...][None, :] \
        + bt_ref[...][None, :]
    h1 = jnp.where(mask, jnp.maximum(h1, 0.0), 0.0)
    hw2 = jnp.dot(h1, w2_ref[...], preferred_element_type=jnp.float32)
    g2_ref[...] = hw2 * dinv[:, None]


def _tc3_body(aggp_ref, g2_ref, degp_ref, b_ref, gm_ref, bt_ref, out_ref):
    dinv, mask = _dinv_and_mask(degp_ref[...])
    agg = aggp_ref[0] + aggp_ref[1]
    pre = (agg + g2_ref[...]) * dinv[:, None] + b_ref[...][None, :]
    cnt = jnp.float32(N)
    mean = jnp.sum(jnp.where(mask, pre, 0.0), axis=0, keepdims=True) / cnt
    dev = jnp.where(mask, pre - mean, 0.0)
    var = jnp.sum(dev * dev, axis=0, keepdims=True) / cnt
    out_ref[...] = (pre - mean) * lax.rsqrt(var + 1e-5) * gm_ref[...][None, :] \
        + bt_ref[...][None, :]


def kernel(x, edge_index, weight, emb_table, W1, b1, gamma1, beta1,
           W2, b2, gamma2, beta2):
    x = x.astype(jnp.int32)
    row = edge_index[0].astype(jnp.int32)
    col = edge_index[1].astype(jnp.int32)
    e = row.shape[0]
    per_w = -(-e // NW)
    ech = -(-per_w // EK)
    epad = NW * ech * EK
    rowp = jnp.pad(row, (0, epad - e)).reshape(NW, ech, EK)
    colp = jnp.pad(col, (0, epad - e)).reshape(NW, ech, EK)
    ewp = jnp.pad(weight, (0, epad - e)).reshape(NW, ech, EK)
    xp = jnp.pad(x, (0, NPAD - N)).reshape(NW, XCH, XK)

    degp, h0 = _make_prep(ech)(colp, ewp, xp, emb_table)

    g1 = pl.pallas_call(
        _tc1_body,
        out_shape=jax.ShapeDtypeStruct((NPAD, 128), jnp.float32),
    )(degp, h0, W1)

    agg1p = _make_agg(ech, 128)(g1, rowp, colp, ewp)

    g2 = pl.pallas_call(
        _tc2_body,
        out_shape=jax.ShapeDtypeStruct((NPAD, 64), jnp.float32),
    )(agg1p, g1, degp, b1, gamma1, beta1, W2)

    agg2p = _make_agg(ech, 64)(g2, rowp, colp, ewp)

    out = pl.pallas_call(
        _tc3_body,
        out_shape=jax.ShapeDtypeStruct((NPAD, 64), jnp.float32),
    )(agg2p, g2, degp, b2, gamma2, beta2)

    return out[:N]
